# baseline (speedup 1.0000x reference)
import functools

import jax
import jax.numpy as jnp
from jax import lax
from jax.experimental import pallas as pl
from jax.experimental.pallas import tpu as pltpu
from jax.experimental.pallas import tpu_sc as plsc

N = 10000
E = 320000
TN = 1000
NT = N // TN
NC, NS = 2, 16
NW = NC * NS
EPW = E // NW
SLICE = N // NS
ZR = 25
F32 = jnp.float32


def _lrelu(x):
    return jnp.maximum(x, 0.2 * x)


def _elu(x):
    return jnp.where(x > 0, x, jnp.exp(jnp.minimum(x, 0.0)) - 1.0)


def _mm_first_body(x_ref, w_ref, asw_ref, adw_ref, h_ref, sd_ref, t_ref, mx_ref):
    _mm_common(x_ref[...], w_ref, asw_ref, adw_ref, h_ref, sd_ref, t_ref, mx_ref)


def _mm_common(x_t, w_ref, asw_ref, adw_ref, h_ref, sd_ref, t_ref, mx_ref):
    i = pl.program_id(0)
    h_t = jnp.dot(x_t, w_ref[...], preferred_element_type=F32)
    h_ref[...] = h_t
    heads, out_ch = asw_ref.shape
    asrc_cols, adst_cols = [], []
    for hd in range(heads):
        hs = h_t[:, hd * out_ch:(hd + 1) * out_ch]
        asrc_cols.append(jnp.sum(hs * asw_ref[hd, :][None, :], axis=1, keepdims=True))
        adst_cols.append(jnp.sum(hs * adw_ref[hd, :][None, :], axis=1, keepdims=True))
    asrc_t = jnp.concatenate(asrc_cols, axis=1)
    adst_t = jnp.concatenate(adst_cols, axis=1)
    sd_ref[...] = jnp.concatenate([asrc_t, adst_t], axis=1)

    @pl.when(i == 0)
    def _():
        mx_ref[...] = jnp.full(mx_ref.shape, -1e30, F32)

    mx_ref[...] = jnp.maximum(mx_ref[...], jnp.max(asrc_t, axis=0, keepdims=True))
    t_ref[...] = mx_ref[...]


def _mm_next_body(p0_ref, p1_ref, hp_ref, s0_ref, s1_ref, ssf_ref, bp_ref,
                  w_ref, asw_ref, adw_ref, h_ref, sd_ref, t_ref, mx_ref,
                  *, heads_prev, chp):
    xs = []
    for hd in range(heads_prev):
        sl = slice(hd * chp, (hd + 1) * chp)
        ss = ssf_ref[:, hd:hd + 1]
        num = p0_ref[:, sl] + p1_ref[:, sl] + ss * hp_ref[:, sl]
        den = s0_ref[:, hd:hd + 1] + s1_ref[:, hd:hd + 1] + ss + 1e-16
        xs.append(_elu(num / den + bp_ref[:, sl]))
    x_t = jnp.concatenate(xs, axis=1)
    _mm_common(x_t, w_ref, asw_ref, adw_ref, h_ref, sd_ref, t_ref, mx_ref)


def _run_mm_first(x, W, asw, adw):
    heads, out_ch = asw.shape
    d_in = x.shape[1]
    d_out = heads * out_ch
    return pl.pallas_call(
        _mm_first_body,
        grid=(NT,),
        in_specs=[
            pl.BlockSpec((TN, d_in), lambda i: (i, 0)),
            pl.BlockSpec((d_in, d_out), lambda i: (0, 0)),
            pl.BlockSpec((heads, out_ch), lambda i: (0, 0)),
            pl.BlockSpec((heads, out_ch), lambda i: (0, 0)),
        ],
        out_specs=[
            pl.BlockSpec((TN, d_out), lambda i: (i, 0)),
            pl.BlockSpec((TN, 2 * heads), lambda i: (i, 0)),
            pl.BlockSpec((1, heads), lambda i: (0, 0)),
        ],
        out_shape=[
            jax.ShapeDtypeStruct((N, d_out), F32),
            jax.ShapeDtypeStruct((N, 2 * heads), F32),
            jax.ShapeDtypeStruct((1, heads), F32),
        ],
        scratch_shapes=[pltpu.VMEM((1, heads), F32)],
    )(x, W, asw, adw)


def _run_mm_next(p0, p1, hp, s0, s1, ssf, bp, W, asw, adw, heads_prev, chp):
    heads, out_ch = asw.shape
    d_in = heads_prev * chp
    d_out = heads * out_ch
    body = functools.partial(_mm_next_body, heads_prev=heads_prev, chp=chp)
    return pl.pallas_call(
        body,
        grid=(NT,),
        in_specs=[
            pl.BlockSpec((TN, d_in), lambda i: (i, 0)),
            pl.BlockSpec((TN, d_in), lambda i: (i, 0)),
            pl.BlockSpec((TN, d_in), lambda i: (i, 0)),
            pl.BlockSpec((TN, 16), lambda i: (i, 0)),
            pl.BlockSpec((TN, 16), lambda i: (i, 0)),
            pl.BlockSpec((TN, heads_prev), lambda i: (i, 0)),
            pl.BlockSpec((1, d_in), lambda i: (0, 0)),
            pl.BlockSpec((d_in, d_out), lambda i: (0, 0)),
            pl.BlockSpec((heads, out_ch), lambda i: (0, 0)),
            pl.BlockSpec((heads, out_ch), lambda i: (0, 0)),
        ],
        out_specs=[
            pl.BlockSpec((TN, d_out), lambda i: (i, 0)),
            pl.BlockSpec((TN, 2 * heads), lambda i: (i, 0)),
            pl.BlockSpec((1, heads), lambda i: (0, 0)),
        ],
        out_shape=[
            jax.ShapeDtypeStruct((N, d_out), F32),
            jax.ShapeDtypeStruct((N, 2 * heads), F32),
            jax.ShapeDtypeStruct((1, heads), F32),
        ],
        scratch_shapes=[pltpu.VMEM((1, heads), F32)],
    )(p0, p1, hp, s0, s1, ssf, bp, W, asw, adw)


def _pack_body(sd_ref, t_ref, sp_ref, dp_ref, ssf_ref):
    asrc = sd_ref[:, 0:8]
    adst = sd_ref[:, 8:16]
    mhat = _lrelu(t_ref[...] + adst)
    sp_ref[...] = jnp.concatenate([asrc, asrc], axis=1)
    dp_ref[...] = jnp.concatenate([adst, adst, -mhat, -mhat], axis=1)
    ssf_ref[...] = jnp.exp(_lrelu(asrc + adst) - mhat)


def _run_pack(sd, T):
    return pl.pallas_call(
        _pack_body,
        grid=(NT,),
        in_specs=[
            pl.BlockSpec((TN, 16), lambda i: (i, 0)),
            pl.BlockSpec((1, 8), lambda i: (0, 0)),
        ],
        out_specs=[
            pl.BlockSpec((TN, 16), lambda i: (i, 0)),
            pl.BlockSpec((TN, 32), lambda i: (i, 0)),
            pl.BlockSpec((TN, 8), lambda i: (i, 0)),
        ],
        out_shape=[
            jax.ShapeDtypeStruct((N, 16), F32),
            jax.ShapeDtypeStruct((N, 32), F32),
            jax.ShapeDtypeStruct((N, 8), F32),
        ],
    )(sd, T)


def _pack4_body(sd_ref, t_ref, h4_ref, tbl_ref, ssf_ref):
    asrc = sd_ref[:, 0:1]
    adst = sd_ref[:, 1:2]
    mhat = _lrelu(t_ref[...] + adst)
    z = jnp.zeros((TN, 12), F32)
    tbl_ref[...] = jnp.concatenate([asrc, adst, -mhat, h4_ref[...], z], axis=1)
    ssf_ref[...] = jnp.exp(_lrelu(asrc + adst) - mhat)


def _run_pack4(sd, T, h4):
    return pl.pallas_call(
        _pack4_body,
        grid=(NT,),
        in_specs=[
            pl.BlockSpec((TN, 2), lambda i: (i, 0)),
            pl.BlockSpec((1, 1), lambda i: (0, 0)),
            pl.BlockSpec((TN, 1), lambda i: (i, 0)),
        ],
        out_specs=[
            pl.BlockSpec((TN, 16), lambda i: (i, 0)),
            pl.BlockSpec((TN, 1), lambda i: (i, 0)),
        ],
        out_shape=[
            jax.ShapeDtypeStruct((N, 16), F32),
            jax.ShapeDtypeStruct((N, 1), F32),
        ],
    )(sd, T, h4)


def _make_sc_edge(C, CW, B0, BC):
    n_b0 = EPW // B0
    n_bc = EPW // BC
    nch = 8 * C // CW
    per = C // CW
    mesh = plsc.VectorSubcoreMesh(core_axis_name="c", subcore_axis_name="s",
                                  num_cores=NC, num_subcores=NS)

    def body(src_hbm, dst_hbm, sp_hbm, dp_hbm, h_hbm, s_out, op_out, exp_out,
             spb, dpb, stg, expw, srcb, dstb,
             csrc0, cdst0, rows0, expr0, csrc1, cdst1, rows1, expr1,
             zbuf, sem0, sem1, s_acc, acc):
        cid = lax.axis_index("c")
        sid = lax.axis_index("s")
        wid = sid * NC + cid
        base = wid * EPW
        bufs = ((csrc0, cdst0, rows0, expr0, sem0),
                (csrc1, cdst1, rows1, expr1, sem1))

        @pl.loop(0, ZR)
        def _(r):
            for j in range(CW // 16):
                zbuf[pl.ds(r, 1), pl.ds(j * 16, 16)] = jnp.zeros((1, 16), F32)

        @pl.loop(0, SLICE // ZR)
        def _(z):
            pltpu.sync_copy(zbuf.at[:, 0:16],
                            s_acc.at[pl.ds(sid * SLICE + z * ZR, ZR), :])
        plsc.subcore_barrier()

        lane_lt8 = (lax.iota(jnp.int32, 16) < 8).reshape(1, 16)

        @pl.loop(0, n_b0)
        def _(ib):
            off = base + ib * B0
            pltpu.sync_copy(src_hbm.at[pl.ds(off, B0)], srcb)
            pltpu.sync_copy(dst_hbm.at[pl.ds(off, B0)], dstb)
            pltpu.sync_copy(sp_hbm.at[srcb], spb)
            pltpu.sync_copy(dp_hbm.at[dstb], dpb)

            @pl.loop(0, B0 // 2, unroll=2)
            def _(p):
                e0 = 2 * p
                t0 = spb[pl.ds(e0, 1), :] + dpb[pl.ds(e0, 1), 0:16]
                w0 = jnp.exp(_lrelu(t0) + dpb[pl.ds(e0, 1), 16:32])
                t1 = spb[pl.ds(e0 + 1, 1), :] + dpb[pl.ds(e0 + 1, 1), 0:16]
                w1 = jnp.exp(_lrelu(t1) + dpb[pl.ds(e0 + 1, 1), 16:32])
                stg[pl.ds(e0, 1), :] = w0
                stg[pl.ds(e0 + 1, 1), :] = w1
                expw[pl.ds(p, 1), :] = jnp.where(lane_lt8, w0, w1)

            pltpu.sync_copy(expw, exp_out.at[wid, pl.ds(ib * (B0 // 2), B0 // 2), :])
            pltpu.sync_copy(stg, s_acc.at[dstb], add=True)

        plsc.subcore_barrier()
        pltpu.sync_copy(s_acc.at[pl.ds(sid * SLICE, SLICE), :],
                        s_out.at[cid, pl.ds(sid * SLICE, SLICE), :])

        def prep(i, buf, q):
            csrc, cdst, rows, expr, sem = buf
            off = base + i * BC
            pltpu.sync_copy(src_hbm.at[pl.ds(off, BC)], csrc)
            pltpu.sync_copy(dst_hbm.at[pl.ds(off, BC)], cdst)

            @pl.loop(0, BC // 16, unroll=4)
            def _(j):
                csrc[pl.ds(j * 16, 16)] = csrc[pl.ds(j * 16, 16)] * nch + q

            pltpu.async_copy(h_hbm.at[csrc], rows, sem)
            pltpu.async_copy(exp_out.at[wid, pl.ds(i * (BC // 2), BC // 2), :],
                             expr, sem)

        def consume(buf, q):
            csrc, cdst, rows, expr, sem = buf
            hd = q // per
            pltpu.make_async_copy(h_hbm.at[csrc], rows, sem).wait()
            pltpu.make_async_copy(
                exp_out.at[wid, pl.ds(0, BC // 2), :], expr, sem).wait()

            @pl.loop(0, BC // 2, unroll=4)
            def _(p):
                wrow = expr[pl.ds(p, 1), :]
                v0 = jnp.full((1, 16), wrow[0, hd], F32)
                v1 = jnp.full((1, 16), wrow[0, 8 + hd], F32)
                for j in range(CW // 16):
                    sl = pl.ds(j * 16, 16)
                    rows[pl.ds(2 * p, 1), sl] = rows[pl.ds(2 * p, 1), sl] * v0
                    rows[pl.ds(2 * p + 1, 1), sl] = rows[pl.ds(2 * p + 1, 1), sl] * v1

            pltpu.sync_copy(rows, acc.at[cdst], add=True)

        for q in range(nch):
            @pl.loop(0, SLICE // ZR)
            def _(z):
                pltpu.sync_copy(zbuf, acc.at[pl.ds(sid * SLICE + z * ZR, ZR), :])
            plsc.subcore_barrier()

            prep(0, bufs[0], q)

            @pl.loop(0, n_bc // 2)
            def _(ib):
                i0 = 2 * ib
                prep(i0 + 1, bufs[1], q)
                consume(bufs[0], q)

                @pl.when(i0 + 2 < n_bc)
                def _():
                    prep(i0 + 2, bufs[0], q)
                consume(bufs[1], q)

            plsc.subcore_barrier()
            pltpu.sync_copy(
                acc.at[pl.ds(sid * SLICE, SLICE), :],
                op_out.at[cid, pl.ds(sid * SLICE, SLICE), pl.ds(q * CW, CW)])

    @functools.partial(
        pl.kernel,
        out_type=(jax.ShapeDtypeStruct((NC, N, 16), F32),
                  jax.ShapeDtypeStruct((NC, N, 8 * C), F32),
                  jax.ShapeDtypeStruct((NW, EPW // 2, 16), F32)),
        mesh=mesh,
        compiler_params=pltpu.CompilerParams(use_tc_tiling_on_sc=False),
        scratch_types=[
            pltpu.VMEM((B0, 16), F32),
            pltpu.VMEM((B0, 32), F32),
            pltpu.VMEM((B0, 16), F32),
            pltpu.VMEM((B0 // 2, 16), F32),
            pltpu.VMEM((B0,), jnp.int32),
            pltpu.VMEM((B0,), jnp.int32),
            pltpu.VMEM((BC,), jnp.int32),
            pltpu.VMEM((BC,), jnp.int32),
            pltpu.VMEM((BC, CW), F32),
            pltpu.VMEM((BC // 2, 16), F32),
            pltpu.VMEM((BC,), jnp.int32),
            pltpu.VMEM((BC,), jnp.int32),
            pltpu.VMEM((BC, CW), F32),
            pltpu.VMEM((BC // 2, 16), F32),
            pltpu.VMEM((ZR, CW), F32),
            pltpu.SemaphoreType.DMA,
            pltpu.SemaphoreType.DMA,
            pltpu.VMEM_SHARED((N, 16), F32),
            pltpu.VMEM_SHARED((N, CW), F32),
        ],
    )
    def k(src_hbm, dst_hbm, sp_hbm, dp_hbm, h_hbm, s_out, op_out, exp_out,
          *scr):
        body(src_hbm, dst_hbm, sp_hbm, dp_hbm, h_hbm, s_out, op_out, exp_out,
             *scr)

    return k


_sc_edge_cache = {}


def _sc_edge(C, CW, B0, BC):
    key = (C, CW, B0, BC)
    if key not in _sc_edge_cache:
        _sc_edge_cache[key] = _make_sc_edge(C, CW, B0, BC)
    return _sc_edge_cache[key]


def _make_sc_edge4(B0):
    n_b0 = EPW // B0
    mesh = plsc.VectorSubcoreMesh(core_axis_name="c", subcore_axis_name="s", num_cores=NC, num_subcores=NS)

    @functools.partial(
        pl.kernel,
        out_type=jax.ShapeDtypeStruct((NC, N, 16), F32),
        mesh=mesh,
        compiler_params=pltpu.CompilerParams(use_tc_tiling_on_sc=False),
        scratch_types=[
            pltpu.VMEM((B0, 16), F32),
            pltpu.VMEM((B0, 16), F32),
            pltpu.VMEM((B0, 16), F32),
            pltpu.VMEM((B0,), jnp.int32),
            pltpu.VMEM((B0,), jnp.int32),
            pltpu.VMEM((ZR, 16), F32),
            pltpu.VMEM_SHARED((N, 16), F32),
        ],
    )
    def k(src_hbm, dst_hbm, tbl_hbm, p_out, sb, db, stg, srcb, dstb, zbuf, s_acc):
        cid = lax.axis_index("c")
        sid = lax.axis_index("s")
        base = (sid * NC + cid) * EPW

        @pl.loop(0, ZR)
        def _(r):
            zbuf[pl.ds(r, 1), :] = jnp.zeros((1, 16), F32)

        @pl.loop(0, SLICE // ZR)
        def _(z):
            pltpu.sync_copy(zbuf, s_acc.at[pl.ds(sid * SLICE + z * ZR, ZR), :])
        plsc.subcore_barrier()

        lane0 = (lax.iota(jnp.int32, 16) == 0).reshape(1, 16)
        lane1 = (lax.iota(jnp.int32, 16) == 1).reshape(1, 16)
        ones = jnp.ones((1, 16), F32)
        zeros = jnp.zeros((1, 16), F32)

        @pl.loop(0, n_b0)
        def _(ib):
            off = base + ib * B0
            pltpu.sync_copy(src_hbm.at[pl.ds(off, B0)], srcb)
            pltpu.sync_copy(dst_hbm.at[pl.ds(off, B0)], dstb)
            pltpu.sync_copy(tbl_hbm.at[srcb], sb)
            pltpu.sync_copy(tbl_hbm.at[dstb], db)

            @pl.loop(0, B0)
            def _(e):
                sv = sb[pl.ds(e, 1), :]
                dv = db[pl.ds(e, 1), :]
                q = sv[0, 0] + dv[0, 1]
                t = jnp.maximum(q, 0.2 * q) + dv[0, 2]
                ev = jnp.exp(jnp.full((1, 16), t, F32))
                hv = jnp.full((1, 16), sv[0, 3], F32)
                w = jnp.where(lane0, hv, jnp.where(lane1, ones, zeros))
                stg[pl.ds(e, 1), :] = ev * w

            pltpu.sync_copy(stg, s_acc.at[dstb], add=True)

        plsc.subcore_barrier()
        pltpu.sync_copy(s_acc.at[pl.ds(sid * SLICE, SLICE), :],
                        p_out.at[cid, pl.ds(sid * SLICE, SLICE), :])

    return k


def _sc_edge4():
    if 'l4' not in _sc_edge_cache:
        _sc_edge_cache['l4'] = _make_sc_edge4(200)
    return _sc_edge_cache['l4']


def _fin_body(p_ref, h4_ref, ssf_ref, b4_ref, o_ref):
    ss = ssf_ref[...]
    num = p_ref[0, :, 0:1] + p_ref[1, :, 0:1] + ss * h4_ref[...]
    den = p_ref[0, :, 1:2] + p_ref[1, :, 1:2] + ss + 1e-16
    o_ref[...] = jax.nn.sigmoid(num / den + b4_ref[0, 0])


def _run_fin(p4, h4, ssf4, b4):
    return pl.pallas_call(
        _fin_body,
        grid=(NT,),
        in_specs=[
            pl.BlockSpec((NC, TN, 16), lambda i: (0, i, 0)),
            pl.BlockSpec((TN, 1), lambda i: (i, 0)),
            pl.BlockSpec((TN, 1), lambda i: (i, 0)),
            pl.BlockSpec((1, 1), lambda i: (0, 0)),
        ],
        out_specs=pl.BlockSpec((TN, 1), lambda i: (i, 0)),
        out_shape=jax.ShapeDtypeStruct((N, 1), F32),
    )(p4, h4, ssf4, b4)


def kernel(x, edge_index, W1, a_src1, a_dst1, b1, W2, a_src2, a_dst2, b2,
           W3, a_src3, a_dst3, b3, W4, a_src4, a_dst4, b4):
    src = edge_index[0]
    dst = edge_index[1]

    h1, sd1, T1 = _run_mm_first(x, W1, a_src1, a_dst1)
    sp1, dp1, ss1 = _run_pack(sd1, T1)
    s1, op1, _e1 = _sc_edge(32, 32, 200, 200)(src, dst, sp1, dp1, h1.reshape(N * 8, 32))

    h2, sd2, T2 = _run_mm_next(op1[0], op1[1], h1, s1[0], s1[1], ss1,
                               b1.reshape(1, -1), W2, a_src2, a_dst2, 8, 32)
    sp2, dp2, ss2 = _run_pack(sd2, T2)
    s2, op2, _e2 = _sc_edge(64, 64, 200, 200)(src, dst, sp2, dp2, h2.reshape(N * 8, 64))

    h3, sd3, T3 = _run_mm_next(op2[0], op2[1], h2, s2[0], s2[1], ss2,
                               b2.reshape(1, -1), W3, a_src3, a_dst3, 8, 64)
    sp3, dp3, ss3 = _run_pack(sd3, T3)
    s3, op3, _e3 = _sc_edge(128, 64, 200, 200)(src, dst, sp3, dp3, h3.reshape(N * 16, 64))

    h4, sd4, T4 = _run_mm_next(op3[0], op3[1], h3, s3[0], s3[1], ss3,
                               b3.reshape(1, -1), W4, a_src4, a_dst4, 8, 128)
    tbl4, ss4 = _run_pack4(sd4, T4, h4)
    p4 = _sc_edge4()(src, dst, tbl4)

    return _run_fin(p4, h4, ss4, b4.reshape(1, 1))


# speedup vs baseline: 1.0063x; 1.0063x over previous
import functools

import jax
import jax.numpy as jnp
from jax import lax
from jax.experimental import pallas as pl
from jax.experimental.pallas import tpu as pltpu
from jax.experimental.pallas import tpu_sc as plsc

N = 10000
E = 320000
TN = 1000
NT = N // TN
NC, NS = 2, 16
NW = NC * NS
EPW = E // NW
SLICE = N // NS
ZR = 25
F32 = jnp.float32


def _lrelu(x):
    return jnp.maximum(x, 0.2 * x)


def _elu(x):
    return jnp.where(x > 0, x, jnp.exp(jnp.minimum(x, 0.0)) - 1.0)


def _mm_first_body(x_ref, w_ref, asw_ref, adw_ref, h_ref, sd_ref, t_ref, mx_ref):
    _mm_common(x_ref[...], w_ref, asw_ref, adw_ref, h_ref, sd_ref, t_ref, mx_ref)


def _mm_common(x_t, w_ref, asw_ref, adw_ref, h_ref, sd_ref, t_ref, mx_ref):
    i = pl.program_id(0)
    h_t = jnp.dot(x_t, w_ref[...], preferred_element_type=F32)
    h_ref[...] = h_t
    heads, out_ch = asw_ref.shape
    asrc_cols, adst_cols = [], []
    for hd in range(heads):
        hs = h_t[:, hd * out_ch:(hd + 1) * out_ch]
        asrc_cols.append(jnp.sum(hs * asw_ref[hd, :][None, :], axis=1, keepdims=True))
        adst_cols.append(jnp.sum(hs * adw_ref[hd, :][None, :], axis=1, keepdims=True))
    asrc_t = jnp.concatenate(asrc_cols, axis=1)
    adst_t = jnp.concatenate(adst_cols, axis=1)
    sd_ref[...] = jnp.concatenate([asrc_t, adst_t], axis=1)

    @pl.when(i == 0)
    def _():
        mx_ref[...] = jnp.full(mx_ref.shape, -1e30, F32)

    mx_ref[...] = jnp.maximum(mx_ref[...], jnp.max(asrc_t, axis=0, keepdims=True))
    t_ref[...] = mx_ref[...]


def _mm_next_body(p0_ref, p1_ref, hp_ref, s0_ref, s1_ref, ssf_ref, bp_ref,
                  w_ref, asw_ref, adw_ref, h_ref, sd_ref, t_ref, mx_ref,
                  *, heads_prev, chp):
    xs = []
    for hd in range(heads_prev):
        sl = slice(hd * chp, (hd + 1) * chp)
        ss = ssf_ref[:, hd:hd + 1]
        num = p0_ref[:, sl] + p1_ref[:, sl] + ss * hp_ref[:, sl]
        den = s0_ref[:, hd:hd + 1] + s1_ref[:, hd:hd + 1] + ss + 1e-16
        xs.append(_elu(num / den + bp_ref[:, sl]))
    x_t = jnp.concatenate(xs, axis=1)
    _mm_common(x_t, w_ref, asw_ref, adw_ref, h_ref, sd_ref, t_ref, mx_ref)


def _run_mm_first(x, W, asw, adw):
    heads, out_ch = asw.shape
    d_in = x.shape[1]
    d_out = heads * out_ch
    return pl.pallas_call(
        _mm_first_body,
        grid=(NT,),
        in_specs=[
            pl.BlockSpec((TN, d_in), lambda i: (i, 0)),
            pl.BlockSpec((d_in, d_out), lambda i: (0, 0)),
            pl.BlockSpec((heads, out_ch), lambda i: (0, 0)),
            pl.BlockSpec((heads, out_ch), lambda i: (0, 0)),
        ],
        out_specs=[
            pl.BlockSpec((TN, d_out), lambda i: (i, 0)),
            pl.BlockSpec((TN, 2 * heads), lambda i: (i, 0)),
            pl.BlockSpec((1, heads), lambda i: (0, 0)),
        ],
        out_shape=[
            jax.ShapeDtypeStruct((N, d_out), F32),
            jax.ShapeDtypeStruct((N, 2 * heads), F32),
            jax.ShapeDtypeStruct((1, heads), F32),
        ],
        scratch_shapes=[pltpu.VMEM((1, heads), F32)],
    )(x, W, asw, adw)


def _run_mm_next(p0, p1, hp, s0, s1, ssf, bp, W, asw, adw, heads_prev, chp):
    heads, out_ch = asw.shape
    d_in = heads_prev * chp
    d_out = heads * out_ch
    body = functools.partial(_mm_next_body, heads_prev=heads_prev, chp=chp)
    return pl.pallas_call(
        body,
        grid=(NT,),
        in_specs=[
            pl.BlockSpec((TN, d_in), lambda i: (i, 0)),
            pl.BlockSpec((TN, d_in), lambda i: (i, 0)),
            pl.BlockSpec((TN, d_in), lambda i: (i, 0)),
            pl.BlockSpec((TN, 16), lambda i: (i, 0)),
            pl.BlockSpec((TN, 16), lambda i: (i, 0)),
            pl.BlockSpec((TN, heads_prev), lambda i: (i, 0)),
            pl.BlockSpec((1, d_in), lambda i: (0, 0)),
            pl.BlockSpec((d_in, d_out), lambda i: (0, 0)),
            pl.BlockSpec((heads, out_ch), lambda i: (0, 0)),
            pl.BlockSpec((heads, out_ch), lambda i: (0, 0)),
        ],
        out_specs=[
            pl.BlockSpec((TN, d_out), lambda i: (i, 0)),
            pl.BlockSpec((TN, 2 * heads), lambda i: (i, 0)),
            pl.BlockSpec((1, heads), lambda i: (0, 0)),
        ],
        out_shape=[
            jax.ShapeDtypeStruct((N, d_out), F32),
            jax.ShapeDtypeStruct((N, 2 * heads), F32),
            jax.ShapeDtypeStruct((1, heads), F32),
        ],
        scratch_shapes=[pltpu.VMEM((1, heads), F32)],
    )(p0, p1, hp, s0, s1, ssf, bp, W, asw, adw)


def _pack_body(sd_ref, t_ref, sp_ref, dp_ref, ssf_ref):
    asrc = sd_ref[:, 0:8]
    adst = sd_ref[:, 8:16]
    mhat = _lrelu(t_ref[...] + adst)
    sp_ref[...] = jnp.concatenate([asrc, asrc], axis=1)
    dp_ref[...] = jnp.concatenate([adst, adst, -mhat, -mhat], axis=1)
    ssf_ref[...] = jnp.exp(_lrelu(asrc + adst) - mhat)


def _run_pack(sd, T):
    return pl.pallas_call(
        _pack_body,
        grid=(NT,),
        in_specs=[
            pl.BlockSpec((TN, 16), lambda i: (i, 0)),
            pl.BlockSpec((1, 8), lambda i: (0, 0)),
        ],
        out_specs=[
            pl.BlockSpec((TN, 16), lambda i: (i, 0)),
            pl.BlockSpec((TN, 32), lambda i: (i, 0)),
            pl.BlockSpec((TN, 8), lambda i: (i, 0)),
        ],
        out_shape=[
            jax.ShapeDtypeStruct((N, 16), F32),
            jax.ShapeDtypeStruct((N, 32), F32),
            jax.ShapeDtypeStruct((N, 8), F32),
        ],
    )(sd, T)


def _pack4_body(sd_ref, t_ref, h4_ref, tbl_ref, ssf_ref):
    asrc = sd_ref[:, 0:1]
    adst = sd_ref[:, 1:2]
    mhat = _lrelu(t_ref[...] + adst)
    z = jnp.zeros((TN, 12), F32)
    tbl_ref[...] = jnp.concatenate([asrc, adst, -mhat, h4_ref[...], z], axis=1)
    ssf_ref[...] = jnp.exp(_lrelu(asrc + adst) - mhat)


def _run_pack4(sd, T, h4):
    return pl.pallas_call(
        _pack4_body,
        grid=(NT,),
        in_specs=[
            pl.BlockSpec((TN, 2), lambda i: (i, 0)),
            pl.BlockSpec((1, 1), lambda i: (0, 0)),
            pl.BlockSpec((TN, 1), lambda i: (i, 0)),
        ],
        out_specs=[
            pl.BlockSpec((TN, 16), lambda i: (i, 0)),
            pl.BlockSpec((TN, 1), lambda i: (i, 0)),
        ],
        out_shape=[
            jax.ShapeDtypeStruct((N, 16), F32),
            jax.ShapeDtypeStruct((N, 1), F32),
        ],
    )(sd, T, h4)


def _make_sc_edge(C, CW, B0, BC):
    n_b0 = EPW // B0
    n_bc = EPW // BC
    nch = 8 * C // CW
    per = C // CW
    mesh = plsc.VectorSubcoreMesh(core_axis_name="c", subcore_axis_name="s",
                                  num_cores=NC, num_subcores=NS)

    def body(src_hbm, dst_hbm, sp_hbm, dp_hbm, h_hbm, s_out, op_out, exp_out,
             spb, dpb, stg, expw, srcb, dstb,
             csrc0, cdst0, rows0, expr0, csrc1, cdst1, rows1, expr1,
             zbuf, sem0, sem1, s_acc, acc):
        cid = lax.axis_index("c")
        sid = lax.axis_index("s")
        wid = sid * NC + cid
        base = wid * EPW
        bufs = ((csrc0, cdst0, rows0, expr0, sem0),
                (csrc1, cdst1, rows1, expr1, sem1))

        @pl.loop(0, ZR)
        def _(r):
            for j in range(CW // 16):
                zbuf[pl.ds(r, 1), pl.ds(j * 16, 16)] = jnp.zeros((1, 16), F32)

        @pl.loop(0, SLICE // ZR)
        def _(z):
            pltpu.sync_copy(zbuf.at[:, 0:16],
                            s_acc.at[pl.ds(sid * SLICE + z * ZR, ZR), :])
        plsc.subcore_barrier()

        lane_lt8 = (lax.iota(jnp.int32, 16) < 8).reshape(1, 16)

        @pl.loop(0, n_b0)
        def _(ib):
            off = base + ib * B0
            pltpu.sync_copy(src_hbm.at[pl.ds(off, B0)], srcb)
            pltpu.sync_copy(dst_hbm.at[pl.ds(off, B0)], dstb)
            pltpu.sync_copy(sp_hbm.at[srcb], spb)
            pltpu.sync_copy(dp_hbm.at[dstb], dpb)

            @pl.loop(0, B0 // 2)
            def _(p):
                e0 = 2 * p
                t0 = spb[pl.ds(e0, 1), :] + dpb[pl.ds(e0, 1), 0:16]
                w0 = jnp.exp(_lrelu(t0) + dpb[pl.ds(e0, 1), 16:32])
                t1 = spb[pl.ds(e0 + 1, 1), :] + dpb[pl.ds(e0 + 1, 1), 0:16]
                w1 = jnp.exp(_lrelu(t1) + dpb[pl.ds(e0 + 1, 1), 16:32])
                stg[pl.ds(e0, 1), :] = w0
                stg[pl.ds(e0 + 1, 1), :] = w1
                expw[pl.ds(p, 1), :] = jnp.where(lane_lt8, w0, w1)

            pltpu.sync_copy(expw, exp_out.at[wid, pl.ds(ib * (B0 // 2), B0 // 2), :])
            pltpu.sync_copy(stg, s_acc.at[dstb], add=True)

        plsc.subcore_barrier()
        pltpu.sync_copy(s_acc.at[pl.ds(sid * SLICE, SLICE), :],
                        s_out.at[cid, pl.ds(sid * SLICE, SLICE), :])

        def prep(i, buf, q):
            csrc, cdst, rows, expr, sem = buf
            off = base + i * BC
            pltpu.sync_copy(src_hbm.at[pl.ds(off, BC)], csrc)
            pltpu.sync_copy(dst_hbm.at[pl.ds(off, BC)], cdst)

            @pl.loop(0, BC // 16)
            def _(j):
                csrc[pl.ds(j * 16, 16)] = csrc[pl.ds(j * 16, 16)] * nch + q

            pltpu.async_copy(h_hbm.at[csrc], rows, sem)
            pltpu.async_copy(exp_out.at[wid, pl.ds(i * (BC // 2), BC // 2), :],
                             expr, sem)

        def consume(buf, q):
            csrc, cdst, rows, expr, sem = buf
            hd = q // per
            pltpu.make_async_copy(h_hbm.at[csrc], rows, sem).wait()
            pltpu.make_async_copy(
                exp_out.at[wid, pl.ds(0, BC // 2), :], expr, sem).wait()

            @pl.loop(0, BC // 2)
            def _(p):
                wrow = expr[pl.ds(p, 1), :]
                v0 = jnp.full((1, 16), wrow[0, hd], F32)
                v1 = jnp.full((1, 16), wrow[0, 8 + hd], F32)
                for j in range(CW // 16):
                    sl = pl.ds(j * 16, 16)
                    rows[pl.ds(2 * p, 1), sl] = rows[pl.ds(2 * p, 1), sl] * v0
                    rows[pl.ds(2 * p + 1, 1), sl] = rows[pl.ds(2 * p + 1, 1), sl] * v1

            pltpu.sync_copy(rows, acc.at[cdst], add=True)

        for q in range(nch):
            @pl.loop(0, SLICE // ZR)
            def _(z):
                pltpu.sync_copy(zbuf, acc.at[pl.ds(sid * SLICE + z * ZR, ZR), :])
            plsc.subcore_barrier()

            prep(0, bufs[0], q)

            @pl.loop(0, n_bc // 2)
            def _(ib):
                i0 = 2 * ib
                prep(i0 + 1, bufs[1], q)
                consume(bufs[0], q)

                @pl.when(i0 + 2 < n_bc)
                def _():
                    prep(i0 + 2, bufs[0], q)
                consume(bufs[1], q)

            plsc.subcore_barrier()
            pltpu.sync_copy(
                acc.at[pl.ds(sid * SLICE, SLICE), :],
                op_out.at[cid, pl.ds(sid * SLICE, SLICE), pl.ds(q * CW, CW)])

    @functools.partial(
        pl.kernel,
        out_type=(jax.ShapeDtypeStruct((NC, N, 16), F32),
                  jax.ShapeDtypeStruct((NC, N, 8 * C), F32),
                  jax.ShapeDtypeStruct((NW, EPW // 2, 16), F32)),
        mesh=mesh,
        compiler_params=pltpu.CompilerParams(use_tc_tiling_on_sc=False),
        scratch_types=[
            pltpu.VMEM((B0, 16), F32),
            pltpu.VMEM((B0, 32), F32),
            pltpu.VMEM((B0, 16), F32),
            pltpu.VMEM((B0 // 2, 16), F32),
            pltpu.VMEM((B0,), jnp.int32),
            pltpu.VMEM((B0,), jnp.int32),
            pltpu.VMEM((BC,), jnp.int32),
            pltpu.VMEM((BC,), jnp.int32),
            pltpu.VMEM((BC, CW), F32),
            pltpu.VMEM((BC // 2, 16), F32),
            pltpu.VMEM((BC,), jnp.int32),
            pltpu.VMEM((BC,), jnp.int32),
            pltpu.VMEM((BC, CW), F32),
            pltpu.VMEM((BC // 2, 16), F32),
            pltpu.VMEM((ZR, CW), F32),
            pltpu.SemaphoreType.DMA,
            pltpu.SemaphoreType.DMA,
            pltpu.VMEM_SHARED((N, 16), F32),
            pltpu.VMEM_SHARED((N, CW), F32),
        ],
    )
    def k(src_hbm, dst_hbm, sp_hbm, dp_hbm, h_hbm, s_out, op_out, exp_out,
          *scr):
        body(src_hbm, dst_hbm, sp_hbm, dp_hbm, h_hbm, s_out, op_out, exp_out,
             *scr)

    return k


_sc_edge_cache = {}


def _sc_edge(C, CW, B0, BC):
    key = (C, CW, B0, BC)
    if key not in _sc_edge_cache:
        _sc_edge_cache[key] = _make_sc_edge(C, CW, B0, BC)
    return _sc_edge_cache[key]


def _make_sc_edge4(B0):
    n_b0 = EPW // B0
    mesh = plsc.VectorSubcoreMesh(core_axis_name="c", subcore_axis_name="s", num_cores=NC, num_subcores=NS)

    @functools.partial(
        pl.kernel,
        out_type=jax.ShapeDtypeStruct((NC, N, 16), F32),
        mesh=mesh,
        compiler_params=pltpu.CompilerParams(use_tc_tiling_on_sc=False),
        scratch_types=[
            pltpu.VMEM((B0, 16), F32),
            pltpu.VMEM((B0, 16), F32),
            pltpu.VMEM((B0, 16), F32),
            pltpu.VMEM((B0,), jnp.int32),
            pltpu.VMEM((B0,), jnp.int32),
            pltpu.VMEM((ZR, 16), F32),
            pltpu.VMEM_SHARED((N, 16), F32),
        ],
    )
    def k(src_hbm, dst_hbm, tbl_hbm, p_out, sb, db, stg, srcb, dstb, zbuf, s_acc):
        cid = lax.axis_index("c")
        sid = lax.axis_index("s")
        base = (sid * NC + cid) * EPW

        @pl.loop(0, ZR)
        def _(r):
            zbuf[pl.ds(r, 1), :] = jnp.zeros((1, 16), F32)

        @pl.loop(0, SLICE // ZR)
        def _(z):
            pltpu.sync_copy(zbuf, s_acc.at[pl.ds(sid * SLICE + z * ZR, ZR), :])
        plsc.subcore_barrier()

        lane0 = (lax.iota(jnp.int32, 16) == 0).reshape(1, 16)
        lane1 = (lax.iota(jnp.int32, 16) == 1).reshape(1, 16)
        ones = jnp.ones((1, 16), F32)
        zeros = jnp.zeros((1, 16), F32)

        @pl.loop(0, n_b0)
        def _(ib):
            off = base + ib * B0
            pltpu.sync_copy(src_hbm.at[pl.ds(off, B0)], srcb)
            pltpu.sync_copy(dst_hbm.at[pl.ds(off, B0)], dstb)
            pltpu.sync_copy(tbl_hbm.at[srcb], sb)
            pltpu.sync_copy(tbl_hbm.at[dstb], db)

            @pl.loop(0, B0)
            def _(e):
                sv = sb[pl.ds(e, 1), :]
                dv = db[pl.ds(e, 1), :]
                q = sv[0, 0] + dv[0, 1]
                t = jnp.maximum(q, 0.2 * q) + dv[0, 2]
                ev = jnp.exp(jnp.full((1, 16), t, F32))
                hv = jnp.full((1, 16), sv[0, 3], F32)
                w = jnp.where(lane0, hv, jnp.where(lane1, ones, zeros))
                stg[pl.ds(e, 1), :] = ev * w

            pltpu.sync_copy(stg, s_acc.at[dstb], add=True)

        plsc.subcore_barrier()
        pltpu.sync_copy(s_acc.at[pl.ds(sid * SLICE, SLICE), :],
                        p_out.at[cid, pl.ds(sid * SLICE, SLICE), :])

    return k


def _sc_edge4():
    if 'l4' not in _sc_edge_cache:
        _sc_edge_cache['l4'] = _make_sc_edge4(200)
    return _sc_edge_cache['l4']


def _fin_body(p_ref, h4_ref, ssf_ref, b4_ref, o_ref):
    ss = ssf_ref[...]
    num = p_ref[0, :, 0:1] + p_ref[1, :, 0:1] + ss * h4_ref[...]
    den = p_ref[0, :, 1:2] + p_ref[1, :, 1:2] + ss + 1e-16
    o_ref[...] = jax.nn.sigmoid(num / den + b4_ref[0, 0])


def _run_fin(p4, h4, ssf4, b4):
    return pl.pallas_call(
        _fin_body,
        grid=(NT,),
        in_specs=[
            pl.BlockSpec((NC, TN, 16), lambda i: (0, i, 0)),
            pl.BlockSpec((TN, 1), lambda i: (i, 0)),
            pl.BlockSpec((TN, 1), lambda i: (i, 0)),
            pl.BlockSpec((1, 1), lambda i: (0, 0)),
        ],
        out_specs=pl.BlockSpec((TN, 1), lambda i: (i, 0)),
        out_shape=jax.ShapeDtypeStruct((N, 1), F32),
    )(p4, h4, ssf4, b4)


def kernel(x, edge_index, W1, a_src1, a_dst1, b1, W2, a_src2, a_dst2, b2,
           W3, a_src3, a_dst3, b3, W4, a_src4, a_dst4, b4):
    src = edge_index[0]
    dst = edge_index[1]

    h1, sd1, T1 = _run_mm_first(x, W1, a_src1, a_dst1)
    sp1, dp1, ss1 = _run_pack(sd1, T1)
    s1, op1, _e1 = _sc_edge(32, 32, 200, 200)(src, dst, sp1, dp1, h1.reshape(N * 8, 32))

    h2, sd2, T2 = _run_mm_next(op1[0], op1[1], h1, s1[0], s1[1], ss1,
                               b1.reshape(1, -1), W2, a_src2, a_dst2, 8, 32)
    sp2, dp2, ss2 = _run_pack(sd2, T2)
    s2, op2, _e2 = _sc_edge(64, 64, 200, 200)(src, dst, sp2, dp2, h2.reshape(N * 8, 64))

    h3, sd3, T3 = _run_mm_next(op2[0], op2[1], h2, s2[0], s2[1], ss2,
                               b2.reshape(1, -1), W3, a_src3, a_dst3, 8, 64)
    sp3, dp3, ss3 = _run_pack(sd3, T3)
    s3, op3, _e3 = _sc_edge(128, 64, 200, 200)(src, dst, sp3, dp3, h3.reshape(N * 16, 64))

    h4, sd4, T4 = _run_mm_next(op3[0], op3[1], h3, s3[0], s3[1], ss3,
                               b3.reshape(1, -1), W4, a_src4, a_dst4, 8, 128)
    tbl4, ss4 = _run_pack4(sd4, T4, h4)
    p4 = _sc_edge4()(src, dst, tbl4)

    return _run_fin(p4, h4, ss4, b4.reshape(1, 1))
